# Initial kernel scaffold; baseline (speedup 1.0000x reference)
import jax
import jax.numpy as jnp
from jax.experimental import pallas as pl


def kernel(node_feat, src_idx, cut_time, ngh_idx_l1, ngh_t_l1, ngh_idx_l2, ngh_t_l2, W_lin, b_lin, freq, phase, a0_Wq, a0_Wk, a0_Wv, a0_fc1_w, a0_fc1_b, a0_fc2_w, a0_fc2_b, a1_Wq, a1_Wk, a1_Wv, a1_fc1_w, a1_fc1_b, a1_fc2_w, a1_fc2_b):
    raise NotImplementedError("write your pallas kernel here")


# speedup vs baseline: 1.5110x; 1.5110x over previous
import functools

import jax
import jax.numpy as jnp
import numpy as np
from jax import lax
from jax.experimental import pallas as pl
from jax.experimental.pallas import tpu as pltpu
from jax.experimental.pallas import tpu_sc as plsc

N, B, K, DF, NH, NHEAD = 100000, 512, 20, 128, 128, 2
DM = 2 * NH
DK = DM // NHEAD
M2 = B * K


def _sc_gather(table, idx):
    info = plsc.get_sparse_core_info()
    NC, NS = info.num_cores, info.num_subcores
    NW = NC * NS
    T = idx.shape[0]
    D = table.shape[1]
    per_w = T // NW
    CH = 128
    n_ch = per_w // CH
    mesh = plsc.VectorSubcoreMesh(core_axis_name="c", subcore_axis_name="s")

    @functools.partial(
        pl.kernel, mesh=mesh,
        out_type=jax.ShapeDtypeStruct((T, D), jnp.float32),
        scratch_types=[
            pltpu.VMEM((2, CH), jnp.int32),
            pltpu.VMEM((2, CH, D), jnp.float32),
            pltpu.SemaphoreType.DMA,
            pltpu.SemaphoreType.DMA,
            pltpu.SemaphoreType.DMA,
        ],
    )
    def k(table_hbm, idx_hbm, out_hbm, idx_v, rows_v, sem_i, sem_g, sem_o):
        wid = lax.axis_index("s") * NC + lax.axis_index("c")
        base0 = wid * per_w

        def idx_load(c, slot):
            return pltpu.async_copy(
                idx_hbm.at[pl.ds(base0 + c * CH, CH)], idx_v.at[slot], sem_i)

        idx_load(0, 0).wait()
        pltpu.async_copy(table_hbm.at[idx_v.at[0]], rows_v.at[0], sem_g).wait()

        def body(c, _):
            slot = lax.rem(c, 2)
            nxt = 1 - slot
            @pl.when(c + 1 < n_ch)
            def _():
                idx_load(c + 1, nxt).wait()
                pltpu.async_copy(table_hbm.at[idx_v.at[nxt]],
                                 rows_v.at[nxt], sem_g)
            pltpu.async_copy(rows_v.at[slot],
                             out_hbm.at[pl.ds(base0 + c * CH, CH)],
                             sem_o).wait()
            @pl.when(c + 1 < n_ch)
            def _():
                pltpu.make_async_copy(table_hbm.at[idx_v.at[nxt]],
                                      rows_v.at[nxt], sem_g).wait()
            return _

        lax.fori_loop(0, n_ch, body, None)

    return k(table, idx)


def _attn_math(src_conv, seq, dt, nidx, freq3, phase3,
               Wq, Wk, Wv, f1w, f1b, f2w, f2b, W_lin=None, b_lin=None):
    BM = src_conv.shape[0]
    tenc0 = jnp.cos(phase3[0])
    q = jnp.concatenate(
        [src_conv, jnp.broadcast_to(tenc0, (BM, NH))], axis=1)
    Q = jnp.dot(q, Wq, preferred_element_type=jnp.float32)
    tenc3 = jnp.cos(dt[:, :, None] * freq3 + phase3)
    mask = nidx == 0
    scale = 1.0 / np.sqrt(DK)
    outs = []
    for h in range(NHEAD):
        Qh = Q[:, h * DK:(h + 1) * DK]
        Wkh = Wk[:, h * DK:(h + 1) * DK]
        Qt = lax.dot_general(Qh, Wkh, (((1,), (1,)), ((), ())),
                             preferred_element_type=jnp.float32)
        Qt_f, Qt_t = Qt[:, :NH], Qt[:, NH:]
        if W_lin is not None:
            Qr = lax.dot_general(Qt_f, W_lin, (((1,), (1,)), ((), ())),
                                 preferred_element_type=jnp.float32)
            sb = lax.dot_general(Qt_f, b_lin, (((1,), (1,)), ((), ())),
                                 preferred_element_type=jnp.float32)
        else:
            Qr = Qt_f
            sb = None
        s = (jnp.sum(seq * Qr[:, None, :], axis=2)
             + jnp.sum(tenc3 * Qt_t[:, None, :], axis=2))
        if sb is not None:
            s = s + sb
        s = s * scale
        s = jnp.where(mask, -1e10, s)
        smax = jnp.max(s, axis=1, keepdims=True)
        e = jnp.exp(s - smax)
        a = e / jnp.sum(e, axis=1, keepdims=True)
        cr = jnp.sum(seq * a[:, :, None], axis=1)
        ct = jnp.sum(tenc3 * a[:, :, None], axis=1)
        Wvh = Wv[:, h * DK:(h + 1) * DK]
        if W_lin is not None:
            cr = jnp.dot(cr, W_lin,
                         preferred_element_type=jnp.float32) + b_lin
        oh = (jnp.dot(cr, Wvh[:NH, :], preferred_element_type=jnp.float32)
              + jnp.dot(ct, Wvh[NH:, :], preferred_element_type=jnp.float32))
        outs.append(oh)
    o = jnp.concatenate(outs, axis=1)
    x = jnp.concatenate([o, src_conv], axis=1)
    h1 = jax.nn.relu(jnp.dot(x, f1w, preferred_element_type=jnp.float32)
                     + f1b)
    return jnp.dot(h1, f2w, preferred_element_type=jnp.float32) + f2b


def _big_body(src_raw_ref, seq_ref, st_ref, nt_ref, nidx_ref,
              wlin_ref, blin_ref, freq_ref, phase_ref,
              wq_ref, wk_ref, wv_ref, f1w_ref, f1b_ref, f2w_ref, f2b_ref,
              out_ref):
    src_conv = (jnp.dot(src_raw_ref[...], wlin_ref[...],
                        preferred_element_type=jnp.float32)
                + blin_ref[...])
    dt = st_ref[...] - nt_ref[...]
    out_ref[...] = _attn_math(
        src_conv, seq_ref[...], dt, nidx_ref[...],
        freq_ref[...], phase_ref[...],
        wq_ref[...], wk_ref[...], wv_ref[...],
        f1w_ref[...], f1b_ref[...], f2w_ref[...], f2b_ref[...],
        W_lin=wlin_ref[...], b_lin=blin_ref[...])


def _attn_big(src_raw, seq, src_t, nt, nidx, W_lin, b_lin2, freq3, phase3,
              Wq, Wk, Wv, f1w, f1b2, f2w, f2b2, bm):
    m = src_raw.shape[0]
    grid = (m // bm,)
    row = lambda i: (i, 0)
    row3 = lambda i: (i, 0, 0)
    fixed = lambda i: (0, 0)
    fixed3 = lambda i: (0, 0, 0)
    return pl.pallas_call(
        _big_body,
        grid=grid,
        in_specs=[
            pl.BlockSpec((bm, NH), row),
            pl.BlockSpec((bm, K, NH), row3),
            pl.BlockSpec((bm, 1), row),
            pl.BlockSpec((bm, K), row),
            pl.BlockSpec((bm, K), row),
            pl.BlockSpec((DF, NH), fixed),
            pl.BlockSpec((1, NH), fixed),
            pl.BlockSpec((1, 1, NH), fixed3),
            pl.BlockSpec((1, 1, NH), fixed3),
            pl.BlockSpec((DM, DM), fixed),
            pl.BlockSpec((DM, DM), fixed),
            pl.BlockSpec((DM, DM), fixed),
            pl.BlockSpec((DM + NH, NH), fixed),
            pl.BlockSpec((1, NH), fixed),
            pl.BlockSpec((NH, NH), fixed),
            pl.BlockSpec((1, NH), fixed),
        ],
        out_specs=pl.BlockSpec((bm, NH), row),
        out_shape=jax.ShapeDtypeStruct((m, NH), jnp.float32),
    )(src_raw, seq, src_t, nt, nidx, W_lin, b_lin2, freq3, phase3,
      Wq, Wk, Wv, f1w, f1b2, f2w, f2b2)


def _small_body(src_raw_ref, seq1_ref, seq2_ref, ct_ref, nt_ref, nidx_ref,
                wlin_ref, blin_ref, freq_ref, phase_ref,
                wq0_ref, wk0_ref, wv0_ref, f1w0_ref, f1b0_ref, f2w0_ref,
                f2b0_ref,
                wq1_ref, wk1_ref, wv1_ref, f1w1_ref, f1b1_ref, f2w1_ref,
                f2b1_ref,
                out_ref):
    src_conv = (jnp.dot(src_raw_ref[...], wlin_ref[...],
                        preferred_element_type=jnp.float32)
                + blin_ref[...])
    dt = ct_ref[...] - nt_ref[...]
    freq3, phase3 = freq_ref[...], phase_ref[...]
    nidx = nidx_ref[...]
    src_l1 = _attn_math(
        src_conv, seq1_ref[...], dt, nidx, freq3, phase3,
        wq0_ref[...], wk0_ref[...], wv0_ref[...],
        f1w0_ref[...], f1b0_ref[...], f2w0_ref[...], f2b0_ref[...],
        W_lin=wlin_ref[...], b_lin=blin_ref[...])
    out_ref[...] = _attn_math(
        src_l1, seq2_ref[...], dt, nidx, freq3, phase3,
        wq1_ref[...], wk1_ref[...], wv1_ref[...],
        f1w1_ref[...], f1b1_ref[...], f2w1_ref[...], f2b1_ref[...])


def _attn_small(src_raw, seq1, seq2, cut_t, nt, nidx,
                W_lin, b_lin2, freq3, phase3, w0, w1, bm):
    m = src_raw.shape[0]
    grid = (m // bm,)
    row = lambda i: (i, 0)
    row3 = lambda i: (i, 0, 0)
    fixed = lambda i: (0, 0)
    fixed3 = lambda i: (0, 0, 0)
    wspecs = [
        pl.BlockSpec((DM, DM), fixed),
        pl.BlockSpec((DM, DM), fixed),
        pl.BlockSpec((DM, DM), fixed),
        pl.BlockSpec((DM + NH, NH), fixed),
        pl.BlockSpec((1, NH), fixed),
        pl.BlockSpec((NH, NH), fixed),
        pl.BlockSpec((1, NH), fixed),
    ]
    return pl.pallas_call(
        _small_body,
        grid=grid,
        in_specs=[
            pl.BlockSpec((bm, NH), row),
            pl.BlockSpec((bm, K, NH), row3),
            pl.BlockSpec((bm, K, NH), row3),
            pl.BlockSpec((bm, 1), row),
            pl.BlockSpec((bm, K), row),
            pl.BlockSpec((bm, K), row),
            pl.BlockSpec((DF, NH), fixed),
            pl.BlockSpec((1, NH), fixed),
            pl.BlockSpec((1, 1, NH), fixed3),
            pl.BlockSpec((1, 1, NH), fixed3),
        ] + wspecs + wspecs,
        out_specs=pl.BlockSpec((bm, NH), row),
        out_shape=jax.ShapeDtypeStruct((m, NH), jnp.float32),
    )(src_raw, seq1, seq2, cut_t, nt, nidx, W_lin, b_lin2, freq3, phase3,
      *w0, *w1)


def kernel(node_feat, src_idx, cut_time, ngh_idx_l1, ngh_t_l1, ngh_idx_l2,
           ngh_t_l2, W_lin, b_lin, freq, phase, a0_Wq, a0_Wk, a0_Wv,
           a0_fc1_w, a0_fc1_b, a0_fc2_w, a0_fc2_b, a1_Wq, a1_Wk, a1_Wv,
           a1_fc1_w, a1_fc1_b, a1_fc2_w, a1_fc2_b):
    n_l1 = B * K
    n_src = B
    n_l2 = B * K * K
    total = n_l1 + n_src + n_l2
    pad_to = 32 * 128
    t_pad = ((total + pad_to - 1) // pad_to) * pad_to
    all_idx = jnp.concatenate([
        ngh_idx_l1.reshape(-1).astype(jnp.int32),
        src_idx.astype(jnp.int32),
        ngh_idx_l2.reshape(-1).astype(jnp.int32),
        jnp.zeros((t_pad - total,), jnp.int32),
    ])
    gathered = _sc_gather(node_feat, all_idx)
    g_l1 = gathered[:n_l1]
    g_src = gathered[n_l1:n_l1 + n_src]
    g_l2 = gathered[n_l1 + n_src:total].reshape(M2, K, NH)

    b_lin2 = b_lin.reshape(1, NH)
    freq3 = freq.reshape(1, 1, NH)
    phase3 = phase.reshape(1, 1, NH)
    f1b0 = a0_fc1_b.reshape(1, NH)
    f2b0 = a0_fc2_b.reshape(1, NH)
    f1b1 = a1_fc1_b.reshape(1, NH)
    f2b1 = a1_fc2_b.reshape(1, NH)

    ngh_l1 = _attn_big(
        g_l1, g_l2, ngh_t_l1.reshape(M2, 1), ngh_t_l2,
        ngh_idx_l2.astype(jnp.int32), W_lin, b_lin2, freq3, phase3,
        a0_Wq, a0_Wk, a0_Wv, a0_fc1_w, f1b0, a0_fc2_w, f2b0, bm=256)

    w0 = (a0_Wq, a0_Wk, a0_Wv, a0_fc1_w, f1b0, a0_fc2_w, f2b0)
    w1 = (a1_Wq, a1_Wk, a1_Wv, a1_fc1_w, f1b1, a1_fc2_w, f2b1)
    out = _attn_small(
        g_src, g_l1.reshape(B, K, NH), ngh_l1.reshape(B, K, NH),
        cut_time.reshape(B, 1), ngh_t_l1, ngh_idx_l1.astype(jnp.int32),
        W_lin, b_lin2, freq3, phase3, w0, w1, bm=128)
    return out
